# Initial kernel scaffold; baseline (speedup 1.0000x reference)
import jax
import jax.numpy as jnp
from jax.experimental import pallas as pl


def kernel(x, edge, batch):
    raise NotImplementedError("write your pallas kernel here")


# speedup vs baseline: 8.0136x; 8.0136x over previous
import functools

import jax
import jax.numpy as jnp
from jax import lax
from jax.experimental import pallas as pl
from jax.experimental.pallas import tpu as pltpu
from jax.experimental.pallas import tpu_sc as plsc

N_ROWS = 100000
N_PAD = 100096
D = 128
NSEG = 256
NW = 32
SEG_PER_W = NSEG // NW
R = 128
LANES = 16
VPR = D // LANES


def _lower_bound(batch_v, t):
    def body(_, c):
        lo, hi = c
        active = lo < hi
        mid = (lo + hi) // 2
        v = batch_v[pl.ds(mid, LANES)][0]
        lt = jnp.logical_and(active, v < t)
        ge = jnp.logical_and(active, v >= t)
        return (jnp.where(lt, mid + 1, lo), jnp.where(ge, mid, hi))

    lo, _ = lax.fori_loop(0, 17, body, (jnp.int32(0), jnp.int32(N_ROWS)))
    return lo


def _sc_body(x_hbm, batch_hbm, out_hbm, batch_v, xbuf, obuf):
    wid = lax.axis_index("s") * 2 + lax.axis_index("c")

    pltpu.sync_copy(batch_hbm, batch_v)

    def accum_rows(base, nrows, accs):
        def row_body(i, accs):
            nums, dens = accs
            off = (base + i) * D
            new_n = []
            new_d = []
            for j in range(VPR):
                xv = xbuf[pl.ds(off + LANES * j, LANES)]
                wv = jnp.exp(xv)
                new_n.append(nums[j] + xv * wv)
                new_d.append(dens[j] + wv)
            return (tuple(new_n), tuple(new_d))

        return lax.fori_loop(0, nrows, row_body, accs)

    def seg_body(k, start):
        s = wid * SEG_PER_W + k
        end = _lower_bound(batch_v, s + 1)
        n = end - start

        zero = jnp.zeros((LANES,), jnp.float32)
        accs = (tuple(zero for _ in range(VPR)), tuple(zero for _ in range(VPR)))

        nfull = n // R

        def blk_body(b, accs):
            r0 = start + b * R
            pltpu.sync_copy(x_hbm.at[pl.ds(r0 * D, R * D)], xbuf)
            return accum_rows(0, R, accs)

        accs = lax.fori_loop(0, nfull, blk_body, accs)

        rem = n - nfull * R
        tail = start + nfull * R
        src = jnp.maximum(end - R, 0)

        pltpu.sync_copy(x_hbm.at[pl.ds(src * D, R * D)], xbuf)
        accs = accum_rows(tail - src, rem, accs)

        nums, dens = accs
        for j in range(VPR):
            obuf[pl.ds(LANES * j, LANES)] = nums[j] / dens[j]
        pltpu.sync_copy(obuf, out_hbm.at[pl.ds(s * D, D)])
        return end

    start0 = _lower_bound(batch_v, wid * SEG_PER_W)
    lax.fori_loop(0, SEG_PER_W, seg_body, start0)


@jax.jit
def kernel(x, edge, batch):
    del edge
    x1 = x.reshape(-1)
    batch_p = jnp.concatenate(
        [batch, jnp.full((N_PAD - N_ROWS,), jnp.int32(1 << 30), jnp.int32)]
    )
    mesh = plsc.VectorSubcoreMesh(core_axis_name="c", subcore_axis_name="s")
    out = pl.kernel(
        _sc_body,
        out_type=jax.ShapeDtypeStruct((NSEG * D,), jnp.float32),
        mesh=mesh,
        scratch_types=[
            pltpu.VMEM((N_PAD,), jnp.int32),
            pltpu.VMEM((R * D,), jnp.float32),
            pltpu.VMEM((D,), jnp.float32),
        ],
    )(x1, batch_p)
    return out.reshape(NSEG, D)


# speedup vs baseline: 11.2544x; 1.4044x over previous
import jax
import jax.numpy as jnp
from jax import lax
from jax.experimental import pallas as pl
from jax.experimental.pallas import tpu as pltpu
from jax.experimental.pallas import tpu_sc as plsc

N_ROWS = 100000
N_PAD = 100096
D = 128
NSEG = 256
NW = 32
SPW = NSEG // NW
R = 128
LANES = 16
VPR = D // LANES
BUF = R * D
OROW = 2 * D


def _lower_bound(batch_v, t):
    def body(_, c):
        lo, hi = c
        active = lo < hi
        mid = (lo + hi) // 2
        v = batch_v[pl.ds(mid, LANES)][0]
        lt = jnp.logical_and(active, v < t)
        ge = jnp.logical_and(active, v >= t)
        return (jnp.where(lt, mid + 1, lo), jnp.where(ge, mid, hi))

    lo, _ = lax.fori_loop(0, 17, body, (jnp.int32(0), jnp.int32(N_ROWS)))
    return lo


def _sc_body(x_hbm, batch_hbm, out_hbm, obuf, acc_st, bounds_sm, sem0, sem1):
    wid = lax.axis_index("s") * 2 + lax.axis_index("c")

    def stage_bounds(batch_v):
        pltpu.sync_copy(batch_hbm, batch_v)
        for k in range(SPW + 1):
            bounds_sm[k] = _lower_bound(batch_v, wid * SPW + k)

    pl.run_scoped(stage_bounds, pltpu.VMEM((N_PAD,), jnp.int32))

    zero = jnp.zeros((LANES,), jnp.float32)
    for r in range(SPW + 1):
        for j in range(2 * VPR):
            acc_st[pl.ds(r * OROW + LANES * j, LANES)] = zero

    ws = bounds_sm[0]
    we = bounds_sm[SPW]
    nblk = (we - ws + R - 1) // R
    npair = (nblk + 1) // 2

    def bsrc(m):
        return jnp.minimum(ws + m * R, N_ROWS - R)

    def main(xbuf):
        def issue(m, base, sem):
            pltpu.async_copy(
                x_hbm.at[pl.ds(bsrc(m) * D, BUF)], xbuf.at[pl.ds(base, BUF)], sem
            )

        def wait(base, sem):
            pltpu.make_async_copy(
                x_hbm.at[pl.ds(0, BUF)], xbuf.at[pl.ds(base, BUF)], sem
            ).wait()

        def process_block(m, base, carry):
            accs, k = carry
            r0 = jnp.minimum(ws + m * R, we)
            r1 = jnp.minimum(ws + (m + 1) * R, we)
            src = bsrc(m)

            def slot(q, c):
                accs, k, kinc = c
                kq = k + q
                valid = kq < SPW
                lo_i = jnp.minimum(kq, SPW - 1)
                seg_lo = bounds_sm[lo_i]
                seg_hi = bounds_sm[lo_i + 1]
                lo = jnp.maximum(r0, seg_lo)
                hi = jnp.minimum(r1, seg_hi)
                cnt = jnp.where(valid, jnp.maximum(hi - lo, 0), 0)

                def row_body(i, accs):
                    nums, dens = accs
                    o = base + (lo - src + i) * D
                    new_n = []
                    new_d = []
                    for j in range(VPR):
                        xv = xbuf[pl.ds(o + LANES * j, LANES)]
                        wv = jnp.exp(xv)
                        new_n.append(nums[j] + xv * wv)
                        new_d.append(dens[j] + wv)
                    return (tuple(new_n), tuple(new_d))

                accs = lax.fori_loop(0, cnt, row_body, accs)

                comp = jnp.logical_and(valid, seg_hi <= r1)
                widx = jnp.minimum(kq, SPW) * OROW
                nums, dens = accs
                keep = 1.0 - comp.astype(jnp.float32)
                new_n = []
                new_d = []
                for j in range(VPR):
                    acc_st[pl.ds(widx + LANES * j, LANES)] = nums[j]
                    acc_st[pl.ds(widx + D + LANES * j, LANES)] = dens[j]
                    new_n.append(nums[j] * keep)
                    new_d.append(dens[j] * keep)
                accs = (tuple(new_n), tuple(new_d))
                return (accs, k, kinc + comp.astype(jnp.int32))

            accs, k, kinc = lax.fori_loop(0, SPW, slot, (accs, k, jnp.int32(0)))
            return (accs, k + kinc)

        issue(0, 0, sem0)

        accs0 = (
            tuple(zero for _ in range(VPR)),
            tuple(zero for _ in range(VPR)),
        )

        def pair_body(p, carry):
            wait(0, sem0)
            issue(2 * p + 1, BUF, sem1)
            carry = process_block(2 * p, 0, carry)
            wait(BUF, sem1)
            issue(2 * p + 2, 0, sem0)
            carry = process_block(2 * p + 1, BUF, carry)
            return carry

        lax.fori_loop(0, npair, pair_body, (accs0, jnp.int32(0)))
        wait(0, sem0)

    pl.run_scoped(main, pltpu.VMEM((2 * BUF,), jnp.float32))

    for r in range(SPW):
        for j in range(VPR):
            nv = acc_st[pl.ds(r * OROW + LANES * j, LANES)]
            dv = acc_st[pl.ds(r * OROW + D + LANES * j, LANES)]
            obuf[pl.ds(r * D + LANES * j, LANES)] = nv / dv
    pltpu.sync_copy(obuf, out_hbm.at[pl.ds(wid * SPW * D, SPW * D)])


@jax.jit
def kernel(x, edge, batch):
    del edge
    x1 = x.reshape(-1)
    batch_p = jnp.concatenate(
        [batch, jnp.full((N_PAD - N_ROWS,), jnp.int32(1 << 30), jnp.int32)]
    )
    mesh = plsc.VectorSubcoreMesh(core_axis_name="c", subcore_axis_name="s")
    out = pl.kernel(
        _sc_body,
        out_type=jax.ShapeDtypeStruct((NSEG * D,), jnp.float32),
        mesh=mesh,
        scratch_types=[
            pltpu.VMEM((SPW * D,), jnp.float32),
            pltpu.VMEM(((SPW + 1) * OROW,), jnp.float32),
            pltpu.SMEM((16,), jnp.int32),
            pltpu.SemaphoreType.DMA,
            pltpu.SemaphoreType.DMA,
        ],
    )(x1, batch_p)
    return out.reshape(NSEG, D)


# speedup vs baseline: 11.3162x; 1.0055x over previous
import jax
import jax.numpy as jnp
from jax import lax
from jax.experimental import pallas as pl
from jax.experimental.pallas import tpu as pltpu
from jax.experimental.pallas import tpu_sc as plsc

N_ROWS = 100000
N_PAD = 100096
D = 128
NSEG = 256
NW = 32
SPW = NSEG // NW
R = 128
LANES = 16
VPR = D // LANES
BUF = R * D
OROW = 2 * D


def _lower_bound(batch_v, t):
    def body(_, c):
        lo, hi = c
        active = lo < hi
        mid = (lo + hi) // 2
        v = batch_v[pl.ds(mid, LANES)][0]
        lt = jnp.logical_and(active, v < t)
        ge = jnp.logical_and(active, v >= t)
        return (jnp.where(lt, mid + 1, lo), jnp.where(ge, mid, hi))

    lo, _ = lax.fori_loop(0, 17, body, (jnp.int32(0), jnp.int32(N_ROWS)))
    return lo


def _sc_body(x_hbm, batch_hbm, out_hbm, obuf, acc_st, bounds_sm, sem0, sem1):
    wid = lax.axis_index("s") * 2 + lax.axis_index("c")

    def stage_bounds(batch_v):
        pltpu.sync_copy(batch_hbm, batch_v)
        for k in range(SPW + 1):
            bounds_sm[k] = _lower_bound(batch_v, wid * SPW + k)

    pl.run_scoped(stage_bounds, pltpu.VMEM((N_PAD,), jnp.int32))

    bscal = [bounds_sm[k] for k in range(SPW + 1)]

    zero = jnp.zeros((LANES,), jnp.float32)
    for r in range(SPW + 1):
        for j in range(2 * VPR):
            acc_st[pl.ds(r * OROW + LANES * j, LANES)] = zero

    ws = bounds_sm[0]
    we = bounds_sm[SPW]
    nblk = (we - ws + R - 1) // R
    npair = (nblk + 1) // 2

    def bsrc(m):
        return jnp.minimum(ws + m * R, N_ROWS - R)

    def main(xbuf):
        def issue(m, base, sem):
            pltpu.async_copy(
                x_hbm.at[pl.ds(bsrc(m) * D, BUF)], xbuf.at[pl.ds(base, BUF)], sem
            )

        def wait(base, sem):
            pltpu.make_async_copy(
                x_hbm.at[pl.ds(0, BUF)], xbuf.at[pl.ds(base, BUF)], sem
            ).wait()

        def process_block(m, base, carry):
            accs, k = carry
            r0 = jnp.minimum(ws + m * R, we)
            r1 = jnp.minimum(ws + (m + 1) * R, we)
            src = bsrc(m)
            ns = jnp.int32(0)
            for k in range(1, SPW + 1):
                ns = ns + jnp.logical_and(bscal[k] > r0, bscal[k] <= r1).astype(
                    jnp.int32
                )

            def slot(q, c):
                accs, k, kinc = c
                kq = k + q
                valid = kq < SPW
                lo_i = jnp.minimum(kq, SPW - 1)
                seg_lo = bounds_sm[lo_i]
                seg_hi = bounds_sm[lo_i + 1]
                lo = jnp.maximum(r0, seg_lo)
                hi = jnp.minimum(r1, seg_hi)
                cnt = jnp.where(valid, jnp.maximum(hi - lo, 0), 0)

                def rows_at(o, accs, nrows):
                    nums, dens = accs
                    new_n = list(nums)
                    new_d = list(dens)
                    for r in range(nrows):
                        for j in range(VPR):
                            xv = xbuf[pl.ds(o + r * D + LANES * j, LANES)]
                            wv = jnp.exp(xv)
                            new_n[j] = new_n[j] + xv * wv
                            new_d[j] = new_d[j] + wv
                    return (tuple(new_n), tuple(new_d))

                def quad_body(i, accs):
                    return rows_at(base + (lo - src + 4 * i) * D, accs, 4)

                def row_body(i, accs):
                    return rows_at(base + (lo - src + i) * D, accs, 1)

                accs = lax.fori_loop(0, cnt // 4, quad_body, accs)
                accs = lax.fori_loop(cnt & ~3, cnt, row_body, accs)

                comp = jnp.logical_and(valid, seg_hi <= r1)
                widx = jnp.minimum(kq, SPW) * OROW
                nums, dens = accs
                keep = 1.0 - comp.astype(jnp.float32)
                new_n = []
                new_d = []
                for j in range(VPR):
                    acc_st[pl.ds(widx + LANES * j, LANES)] = nums[j]
                    acc_st[pl.ds(widx + D + LANES * j, LANES)] = dens[j]
                    new_n.append(nums[j] * keep)
                    new_d.append(dens[j] * keep)
                accs = (tuple(new_n), tuple(new_d))
                return (accs, k, kinc + comp.astype(jnp.int32))

            accs, k, kinc = lax.fori_loop(0, ns + 1, slot, (accs, k, jnp.int32(0)))
            return (accs, k + kinc)

        issue(0, 0, sem0)

        accs0 = (
            tuple(zero for _ in range(VPR)),
            tuple(zero for _ in range(VPR)),
        )

        def pair_body(p, carry):
            wait(0, sem0)
            issue(2 * p + 1, BUF, sem1)
            carry = process_block(2 * p, 0, carry)
            wait(BUF, sem1)
            issue(2 * p + 2, 0, sem0)
            carry = process_block(2 * p + 1, BUF, carry)
            return carry

        lax.fori_loop(0, npair, pair_body, (accs0, jnp.int32(0)))
        wait(0, sem0)

    pl.run_scoped(main, pltpu.VMEM((2 * BUF,), jnp.float32))

    for r in range(SPW):
        for j in range(VPR):
            nv = acc_st[pl.ds(r * OROW + LANES * j, LANES)]
            dv = acc_st[pl.ds(r * OROW + D + LANES * j, LANES)]
            obuf[pl.ds(r * D + LANES * j, LANES)] = nv / dv
    pltpu.sync_copy(obuf, out_hbm.at[pl.ds(wid * SPW * D, SPW * D)])


@jax.jit
def kernel(x, edge, batch):
    del edge
    x1 = x.reshape(-1)
    batch_p = jnp.concatenate(
        [batch, jnp.full((N_PAD - N_ROWS,), jnp.int32(1 << 30), jnp.int32)]
    )
    mesh = plsc.VectorSubcoreMesh(core_axis_name="c", subcore_axis_name="s")
    out = pl.kernel(
        _sc_body,
        out_type=jax.ShapeDtypeStruct((NSEG * D,), jnp.float32),
        mesh=mesh,
        scratch_types=[
            pltpu.VMEM((SPW * D,), jnp.float32),
            pltpu.VMEM(((SPW + 1) * OROW,), jnp.float32),
            pltpu.SMEM((16,), jnp.int32),
            pltpu.SemaphoreType.DMA,
            pltpu.SemaphoreType.DMA,
        ],
    )(x1, batch_p)
    return out.reshape(NSEG, D)
